# baseline (speedup 1.0000x reference)
import functools

import jax
import jax.numpy as jnp
from jax import lax
from jax.experimental import pallas as pl
from jax.experimental.pallas import tpu as pltpu
from jax.experimental.pallas import tpu_sc as plsc

N = 10000
E = 320000
F = 64
NC, NS = 2, 16
NW = NC * NS
B = 128
CH = 80
EP = NW * CH * B
ROWS_PER_TILE = 632
NP = NS * ROWS_PER_TILE


def _segsum_sc(g, colp, rowp):
    mesh = plsc.VectorSubcoreMesh(core_axis_name="c", subcore_axis_name="s")

    @functools.partial(
        pl.kernel,
        out_type=jax.ShapeDtypeStruct((NC, NP, F), jnp.float32),
        mesh=mesh,
        scratch_types=[
            pltpu.VMEM((CH, B), jnp.int32),
            pltpu.VMEM((CH, B), jnp.int32),
            pltpu.VMEM((4, B, F), jnp.float32),
            pltpu.VMEM_SHARED((NP, F), jnp.float32),
            [pltpu.SemaphoreType.DMA] * 4,
        ],
        compiler_params=pltpu.CompilerParams(use_tc_tiling_on_sc=False),
    )
    def k(g_hbm, col_hbm, row_hbm, out_hbm, col_v, row_v, buf, acc,
          sem_g):
        c = lax.axis_index("c")
        s = lax.axis_index("s")
        wid = s * NC + c

        z = jnp.zeros((16,), jnp.float32)

        def zero_row(i, _):
            k = i // B
            r = i % B
            buf[k, r, pl.ds(0, 16)] = z
            buf[k, r, pl.ds(16, 16)] = z
            buf[k, r, pl.ds(32, 16)] = z
            buf[k, r, pl.ds(48, 16)] = z
            return _

        lax.fori_loop(0, 4 * B, zero_row, 0)
        a0 = s * ROWS_PER_TILE
        for zc in range(4):
            pltpu.sync_copy(buf.at[zc], acc.at[pl.ds(a0 + zc * B, B)])
        pltpu.sync_copy(buf.at[0].at[pl.ds(0, ROWS_PER_TILE - 4 * B)],
                        acc.at[pl.ds(a0 + 4 * B, ROWS_PER_TILE - 4 * B)])

        pltpu.sync_copy(col_hbm.at[wid], col_v)
        pltpu.sync_copy(row_hbm.at[wid], row_v)

        plsc.subcore_barrier()

        def fire_gather(j, slot):
            pltpu.async_copy(g_hbm.at[col_v.at[j]], buf.at[slot], sem_g[slot])

        def wait_gather(j, slot):
            pltpu.make_async_copy(
                g_hbm.at[col_v.at[j]], buf.at[slot], sem_g[slot]).wait()

        fire_gather(0, 0)

        def edge_chunk2(j2, carry):
            j = j2 * 2
            for u in range(2):
                ju = j + u
                jf = ju + 1
                kf = (u + 1) % 2

                @pl.when(jf < CH)
                def _(jf=jf, kf=kf):
                    fire_gather(jf, kf)
                wait_gather(ju, u)
                pltpu.sync_copy(buf.at[u], acc.at[row_v.at[ju]], add=True)
            return carry

        lax.fori_loop(0, CH // 2, edge_chunk2, 0)

        plsc.subcore_barrier()

        r0 = s * ROWS_PER_TILE
        for zc in range(5):
            rows = B if zc < 4 else ROWS_PER_TILE - 4 * B
            pltpu.sync_copy(acc.at[pl.ds(r0 + zc * B, rows)],
                            buf.at[zc % 4].at[pl.ds(0, rows)])
            pltpu.sync_copy(buf.at[zc % 4].at[pl.ds(0, rows)],
                            out_hbm.at[c].at[pl.ds(r0 + zc * B, rows)])

    return k(g, colp, rowp)


def _mm_body(x_ref, w_ref, o_ref):
    o_ref[:] = jnp.dot(x_ref[:], w_ref[:], preferred_element_type=jnp.float32)


def _combine_body(p_ref, b_ref, w_ref, o_ref):
    h = jnp.maximum(p_ref[0] + p_ref[1] + b_ref[:], 0.0)
    o_ref[:] = jnp.dot(h, w_ref[:], preferred_element_type=jnp.float32)


def _head_body(p_ref, b3_ref, wf1_ref, bf1_ref, wf2_ref, bf2_ref, o_ref):
    h = jnp.maximum(p_ref[0, :N] + p_ref[1, :N] + b3_ref[:], 0.0)
    m = jnp.sum(h, axis=0, keepdims=True) * (1.0 / N)
    a = jnp.dot(m, wf1_ref[:], preferred_element_type=jnp.float32) + bf1_ref[:]
    a = jnp.maximum(a, 0.0)
    z = jnp.sum(a * wf2_ref[:], axis=1, keepdims=True) + bf2_ref[:]
    o_ref[:] = jax.nn.sigmoid(z)


def kernel(x, edge_index, W1, b1, W2, b2, W3, b3, Wf1, bf1, Wf2, bf2):
    f32 = jnp.float32
    col = edge_index[1]
    row = edge_index[0]
    colp = jnp.pad(col, (0, EP - E)).reshape(NW, CH, B)
    rowp = jnp.pad(row, (0, EP - E), constant_values=N).reshape(NW, CH, B)

    xp = jnp.pad(x, ((0, NP - N), (0, 0)))

    g = pl.pallas_call(
        _mm_body, out_shape=jax.ShapeDtypeStruct((NP, F), f32)
    )(xp, W1.T)

    for W_next, b in ((W2, b1), (W3, b2)):
        p = _segsum_sc(g, colp, rowp)
        g = pl.pallas_call(
            _combine_body, out_shape=jax.ShapeDtypeStruct((NP, F), f32)
        )(p, b.reshape(1, F), W_next.T)

    p = _segsum_sc(g, colp, rowp)
    out = pl.pallas_call(
        _head_body, out_shape=jax.ShapeDtypeStruct((1, 1), f32)
    )(p, b3.reshape(1, F), Wf1.T, bf1.reshape(1, 32), Wf2, bf2.reshape(1, 1))
    return out.reshape(1)


# speedup vs baseline: 1.0021x; 1.0021x over previous
import functools

import jax
import jax.numpy as jnp
from jax import lax
from jax.experimental import pallas as pl
from jax.experimental.pallas import tpu as pltpu
from jax.experimental.pallas import tpu_sc as plsc

N = 10000
E = 320000
F = 64
NC, NS = 2, 16
NW = NC * NS
B = 128
CH = 80
EP = NW * CH * B
ROWS_PER_TILE = 632
NP = NS * ROWS_PER_TILE


def _segsum_sc(g, colp, rowp):
    mesh = plsc.VectorSubcoreMesh(core_axis_name="c", subcore_axis_name="s")

    @functools.partial(
        pl.kernel,
        out_type=jax.ShapeDtypeStruct((NC, NP, F), jnp.float32),
        mesh=mesh,
        scratch_types=[
            pltpu.VMEM((CH, B), jnp.int32),
            pltpu.VMEM((CH, B), jnp.int32),
            pltpu.VMEM((2, B, F), jnp.float32),
            pltpu.VMEM((ROWS_PER_TILE, F), jnp.float32),
            pltpu.VMEM_SHARED((NP, F), jnp.float32),
            [pltpu.SemaphoreType.DMA] * 2,
        ],
        compiler_params=pltpu.CompilerParams(use_tc_tiling_on_sc=False),
    )
    def k(g_hbm, col_hbm, row_hbm, out_hbm, col_v, row_v, buf, stage, acc,
          sem_g):
        c = lax.axis_index("c")
        s = lax.axis_index("s")
        wid = s * NC + c

        z = jnp.zeros((16,), jnp.float32)

        def zero_row(i, _):
            stage[i, pl.ds(0, 16)] = z
            stage[i, pl.ds(16, 16)] = z
            stage[i, pl.ds(32, 16)] = z
            stage[i, pl.ds(48, 16)] = z
            return _

        lax.fori_loop(0, ROWS_PER_TILE, zero_row, 0)
        pltpu.sync_copy(stage, acc.at[pl.ds(s * ROWS_PER_TILE, ROWS_PER_TILE)])

        pltpu.sync_copy(col_hbm.at[wid], col_v)
        pltpu.sync_copy(row_hbm.at[wid], row_v)

        plsc.subcore_barrier()

        def fire_gather(j, slot):
            pltpu.async_copy(g_hbm.at[col_v.at[j]], buf.at[slot], sem_g[slot])

        def wait_gather(j, slot):
            pltpu.make_async_copy(
                g_hbm.at[col_v.at[j]], buf.at[slot], sem_g[slot]).wait()

        fire_gather(0, 0)

        def edge_chunk2(j2, carry):
            j = j2 * 2
            for u in range(2):
                ju = j + u
                jf = ju + 1
                kf = (u + 1) % 2

                @pl.when(jf < CH)
                def _(jf=jf, kf=kf):
                    fire_gather(jf, kf)
                wait_gather(ju, u)
                pltpu.sync_copy(buf.at[u], acc.at[row_v.at[ju]], add=True)
            return carry

        lax.fori_loop(0, CH // 2, edge_chunk2, 0)
        plsc.subcore_barrier()

        r0 = s * ROWS_PER_TILE
        pltpu.sync_copy(acc.at[pl.ds(r0, ROWS_PER_TILE)], stage)
        pltpu.sync_copy(stage, out_hbm.at[c].at[pl.ds(r0, ROWS_PER_TILE)])

    return k(g, colp, rowp)


def _mm_body(x_ref, w_ref, o_ref):
    o_ref[:] = jnp.dot(x_ref[:], w_ref[:], preferred_element_type=jnp.float32)


def _combine_body(p_ref, b_ref, w_ref, o_ref):
    h = jnp.maximum(p_ref[0] + p_ref[1] + b_ref[:], 0.0)
    o_ref[:] = jnp.dot(h, w_ref[:], preferred_element_type=jnp.float32)


def _head_body(p_ref, b3_ref, wf1_ref, bf1_ref, wf2_ref, bf2_ref, o_ref):
    h = jnp.maximum(p_ref[0, :N] + p_ref[1, :N] + b3_ref[:], 0.0)
    m = jnp.sum(h, axis=0, keepdims=True) * (1.0 / N)
    a = jnp.dot(m, wf1_ref[:], preferred_element_type=jnp.float32) + bf1_ref[:]
    a = jnp.maximum(a, 0.0)
    z = jnp.sum(a * wf2_ref[:], axis=1, keepdims=True) + bf2_ref[:]
    o_ref[:] = jax.nn.sigmoid(z)


def kernel(x, edge_index, W1, b1, W2, b2, W3, b3, Wf1, bf1, Wf2, bf2):
    f32 = jnp.float32
    col = edge_index[1]
    row = edge_index[0]
    colp = jnp.pad(col, (0, EP - E)).reshape(NW, CH, B)
    rowp = jnp.pad(row, (0, EP - E), constant_values=N).reshape(NW, CH, B)

    xp = jnp.pad(x, ((0, NP - N), (0, 0)))

    g = pl.pallas_call(
        _mm_body, out_shape=jax.ShapeDtypeStruct((NP, F), f32)
    )(xp, W1.T)

    for W_next, b in ((W2, b1), (W3, b2)):
        p = _segsum_sc(g, colp, rowp)
        g = pl.pallas_call(
            _combine_body, out_shape=jax.ShapeDtypeStruct((NP, F), f32)
        )(p, b.reshape(1, F), W_next.T)

    p = _segsum_sc(g, colp, rowp)
    out = pl.pallas_call(
        _head_body, out_shape=jax.ShapeDtypeStruct((1, 1), f32)
    )(p, b3.reshape(1, F), Wf1.T, bf1.reshape(1, 32), Wf2, bf2.reshape(1, 1))
    return out.reshape(1)


# speedup vs baseline: 1.0749x; 1.0726x over previous
import functools

import jax
import jax.numpy as jnp
from jax import lax
from jax.experimental import pallas as pl
from jax.experimental.pallas import tpu as pltpu
from jax.experimental.pallas import tpu_sc as plsc

N = 10000
E = 320000
F = 64
NC, NS = 2, 16
NW = NC * NS
B = 128
CH = 80
EP = NW * CH * B
ROWS_PER_TILE = 632
NP = NS * ROWS_PER_TILE


def _segsum_sc(g, colp, rowp):
    mesh = plsc.VectorSubcoreMesh(core_axis_name="c", subcore_axis_name="s")

    @functools.partial(
        pl.kernel,
        out_type=jax.ShapeDtypeStruct((NC, NP, F), jnp.float32),
        mesh=mesh,
        scratch_types=[
            pltpu.VMEM((CH, B), jnp.int32),
            pltpu.VMEM((CH, B), jnp.int32),
            pltpu.VMEM((2, B, F), jnp.float32),
            pltpu.VMEM((ROWS_PER_TILE, F), jnp.float32),
            pltpu.VMEM_SHARED((NP, F), jnp.float32),
            [pltpu.SemaphoreType.DMA] * 2,
        ],
        compiler_params=pltpu.CompilerParams(use_tc_tiling_on_sc=False),
    )
    def k(g_hbm, col_hbm, row_hbm, out_hbm, col_v, row_v, buf, stage, acc,
          sem_g):
        c = lax.axis_index("c")
        s = lax.axis_index("s")
        wid = s * NC + c

        z = jnp.zeros((16,), jnp.float32)

        def zero_row(i, _):
            stage[i, pl.ds(0, 16)] = z
            stage[i, pl.ds(16, 16)] = z
            stage[i, pl.ds(32, 16)] = z
            stage[i, pl.ds(48, 16)] = z
            return _

        lax.fori_loop(0, ROWS_PER_TILE, zero_row, 0)
        pltpu.sync_copy(stage, acc.at[pl.ds(s * ROWS_PER_TILE, ROWS_PER_TILE)])

        pltpu.sync_copy(col_hbm.at[wid], col_v)
        pltpu.sync_copy(row_hbm.at[wid], row_v)

        plsc.subcore_barrier()

        def fire_gather(j, slot):
            pltpu.async_copy(g_hbm.at[col_v.at[j]], buf.at[slot], sem_g[slot])

        def wait_gather(j, slot):
            pltpu.make_async_copy(
                g_hbm.at[col_v.at[j]], buf.at[slot], sem_g[slot]).wait()

        fire_gather(0, 0)

        def edge_chunk2(j2, carry):
            j = j2 * 2
            for u in range(2):
                ju = j + u
                jf = ju + 1
                kf = (u + 1) % 2

                @pl.when(jf < CH)
                def _(jf=jf, kf=kf):
                    fire_gather(jf, kf)
                wait_gather(ju, u)
                pltpu.sync_copy(buf.at[u], acc.at[row_v.at[ju]], add=True)
            return carry

        lax.fori_loop(0, CH // 2, edge_chunk2, 0)
        plsc.subcore_barrier()

        r0 = s * ROWS_PER_TILE
        pltpu.sync_copy(acc.at[pl.ds(r0, ROWS_PER_TILE)], stage)
        pltpu.sync_copy(stage, out_hbm.at[c].at[pl.ds(r0, ROWS_PER_TILE)])

    return k(g, colp, rowp)


def _mm_body(x_ref, w_ref, o_ref):
    o_ref[:] = jnp.dot(x_ref[:], w_ref[:], preferred_element_type=jnp.float32)


def _combine_body(p_ref, b_ref, w_ref, o_ref):
    h = jnp.maximum(p_ref[0] + p_ref[1] + b_ref[:], 0.0)
    o_ref[:] = jnp.dot(h, w_ref[:], preferred_element_type=jnp.float32)


def _head_body(p_ref, b3_ref, wf1_ref, bf1_ref, wf2_ref, bf2_ref, o_ref):
    h = jnp.maximum(p_ref[0, :N] + p_ref[1, :N] + b3_ref[:], 0.0)
    m = jnp.sum(h, axis=0, keepdims=True) * (1.0 / N)
    a = jnp.dot(m, wf1_ref[:], preferred_element_type=jnp.float32) + bf1_ref[:]
    a = jnp.maximum(a, 0.0)
    z = jnp.sum(a * wf2_ref[:], axis=1, keepdims=True) + bf2_ref[:]
    o_ref[:] = jax.nn.sigmoid(z)


def kernel(x, edge_index, W1, b1, W2, b2, W3, b3, Wf1, bf1, Wf2, bf2):
    f32 = jnp.float32
    col = edge_index[1]
    row = edge_index[0]
    pad_rows = N + (jnp.arange(EP - E, dtype=jnp.int32) % 109)
    colp = jnp.pad(col, (0, EP - E)).reshape(CH * B, NW).T.reshape(NW, CH, B)
    rowp = (jnp.concatenate([row, pad_rows])
            .reshape(CH * B, NW).T.reshape(NW, CH, B))

    xp = jnp.pad(x, ((0, NP - N), (0, 0)))

    g = pl.pallas_call(
        _mm_body, out_shape=jax.ShapeDtypeStruct((NP, F), f32)
    )(xp, W1.T)

    for W_next, b in ((W2, b1), (W3, b2)):
        p = _segsum_sc(g, colp, rowp)
        g = pl.pallas_call(
            _combine_body, out_shape=jax.ShapeDtypeStruct((NP, F), f32)
        )(p, b.reshape(1, F), W_next.T)

    p = _segsum_sc(g, colp, rowp)
    out = pl.pallas_call(
        _head_body, out_shape=jax.ShapeDtypeStruct((1, 1), f32)
    )(p, b3.reshape(1, F), Wf1.T, bf1.reshape(1, 32), Wf2, bf2.reshape(1, 1))
    return out.reshape(1)


# speedup vs baseline: 1.0937x; 1.0175x over previous
import functools

import jax
import jax.numpy as jnp
from jax import lax
from jax.experimental import pallas as pl
from jax.experimental.pallas import tpu as pltpu
from jax.experimental.pallas import tpu_sc as plsc

N = 10000
E = 320000
F = 64
NC, NS = 2, 16
NW = NC * NS
B = 128
CH = 80
EP = NW * CH * B
ROWS_PER_TILE = 632
NP = NS * ROWS_PER_TILE


def _segsum_sc(g, colp, rowp):
    mesh = plsc.VectorSubcoreMesh(core_axis_name="c", subcore_axis_name="s")

    @functools.partial(
        pl.kernel,
        out_type=jax.ShapeDtypeStruct((NC, NP, F), jnp.float32),
        mesh=mesh,
        scratch_types=[
            pltpu.VMEM((CH, B), jnp.int32),
            pltpu.VMEM((CH, B), jnp.int32),
            pltpu.VMEM((2, B, F), jnp.float32),
            pltpu.VMEM((ROWS_PER_TILE, F), jnp.float32),
            pltpu.VMEM_SHARED((NP, F), jnp.float32),
            [pltpu.SemaphoreType.DMA] * 2,
        ],
        compiler_params=pltpu.CompilerParams(use_tc_tiling_on_sc=False),
    )
    def k(g_hbm, col_hbm, row_hbm, out_hbm, col_v, row_v, buf, stage, acc,
          sem_g):
        c = lax.axis_index("c")
        s = lax.axis_index("s")
        wid = s * NC + c

        z = jnp.zeros((16,), jnp.float32)

        def zero_row(i, _):
            stage[i, pl.ds(0, 16)] = z
            stage[i, pl.ds(16, 16)] = z
            stage[i, pl.ds(32, 16)] = z
            stage[i, pl.ds(48, 16)] = z
            return _

        lax.fori_loop(0, ROWS_PER_TILE, zero_row, 0)
        pltpu.sync_copy(stage, acc.at[pl.ds(s * ROWS_PER_TILE, ROWS_PER_TILE)])

        pltpu.sync_copy(col_hbm.at[wid], col_v)
        pltpu.sync_copy(row_hbm.at[wid], row_v)

        plsc.subcore_barrier()

        def fire_gather(j, slot):
            pltpu.async_copy(g_hbm.at[col_v.at[j]], buf.at[slot], sem_g[slot])

        def wait_gather(j, slot):
            pltpu.make_async_copy(
                g_hbm.at[col_v.at[j]], buf.at[slot], sem_g[slot]).wait()

        fire_gather(0, 0)

        def edge_chunk2(j2, carry):
            j = j2 * 2
            for u in range(2):
                ju = j + u
                jf = ju + 1
                kf = (u + 1) % 2

                @pl.when(jf < CH)
                def _(jf=jf, kf=kf):
                    fire_gather(jf, kf)
                wait_gather(ju, u)
                pltpu.sync_copy(buf.at[u], acc.at[row_v.at[ju]], add=True)
            return carry

        lax.fori_loop(0, CH // 2, edge_chunk2, 0)
        plsc.subcore_barrier()

        r0 = s * ROWS_PER_TILE
        pltpu.sync_copy(acc.at[pl.ds(r0, ROWS_PER_TILE)], stage)
        pltpu.sync_copy(stage, out_hbm.at[c].at[pl.ds(r0, ROWS_PER_TILE)])

    return k(g, colp, rowp)


def _mm_body(x_ref, w_ref, o_ref):
    o_ref[:] = jnp.dot(x_ref[:], w_ref[:], preferred_element_type=jnp.float32)


def _combine_body(p_ref, b_ref, w_ref, o_ref):
    h = jnp.maximum(p_ref[0] + p_ref[1] + b_ref[:], 0.0)
    o_ref[:] = jnp.dot(h, w_ref[:], preferred_element_type=jnp.float32)


def _head_body(p_ref, b3_ref, wf1_ref, bf1_ref, wf2_ref, bf2_ref, o_ref):
    h = jnp.maximum(p_ref[0, :N] + p_ref[1, :N] + b3_ref[:], 0.0)
    m = jnp.sum(h, axis=0, keepdims=True) * (1.0 / N)
    a = jnp.dot(m, wf1_ref[:], preferred_element_type=jnp.float32) + bf1_ref[:]
    a = jnp.maximum(a, 0.0)
    z = jnp.sum(a * wf2_ref[:], axis=1, keepdims=True) + bf2_ref[:]
    o_ref[:] = jax.nn.sigmoid(z)


def kernel(x, edge_index, W1, b1, W2, b2, W3, b3, Wf1, bf1, Wf2, bf2):
    f32 = jnp.float32
    col = edge_index[1]
    row = edge_index[0]
    pad_per_w = CH * B - E // NW
    pad_rows = jnp.broadcast_to(
        N + jnp.arange(pad_per_w, dtype=jnp.int32) % (NP - N), (NW, pad_per_w))
    colp = jnp.pad(col.reshape(NW, E // NW),
                   ((0, 0), (0, pad_per_w))).reshape(NW, CH, B)
    rowp = jnp.concatenate(
        [row.reshape(NW, E // NW), pad_rows], axis=1).reshape(NW, CH, B)

    xp = jnp.pad(x, ((0, NP - N), (0, 0)))

    g = pl.pallas_call(
        _mm_body, out_shape=jax.ShapeDtypeStruct((NP, F), f32)
    )(xp, W1.T)

    for W_next, b in ((W2, b1), (W3, b2)):
        p = _segsum_sc(g, colp, rowp)
        g = pl.pallas_call(
            _combine_body, out_shape=jax.ShapeDtypeStruct((NP, F), f32)
        )(p, b.reshape(1, F), W_next.T)

    p = _segsum_sc(g, colp, rowp)
    out = pl.pallas_call(
        _head_body, out_shape=jax.ShapeDtypeStruct((1, 1), f32)
    )(p, b3.reshape(1, F), Wf1.T, bf1.reshape(1, 32), Wf2, bf2.reshape(1, 1))
    return out.reshape(1)


# speedup vs baseline: 2.3372x; 2.1370x over previous
import functools

import jax
import jax.numpy as jnp
from jax import lax
from jax.experimental import pallas as pl
from jax.experimental.pallas import tpu as pltpu
from jax.experimental.pallas import tpu_sc as plsc

N = 10000
E = 320000
F = 64
NC, NS = 2, 16
NW = NC * NS
B = 128
CH = 80
EP = NW * CH * B
ROWS_PER_TILE = 632
NP = NS * ROWS_PER_TILE


def _segsum_sc(g, colp, rowp):
    mesh = plsc.VectorSubcoreMesh(core_axis_name="c", subcore_axis_name="s")

    @functools.partial(
        pl.kernel,
        out_type=jax.ShapeDtypeStruct((NC, NP, F), jnp.float32),
        mesh=mesh,
        scratch_types=[
            pltpu.VMEM((CH, B), jnp.int32),
            pltpu.VMEM((CH, B), jnp.int32),
            pltpu.VMEM((2, B, F), jnp.float32),
            pltpu.VMEM_SHARED((NP, F), jnp.float32),
            pltpu.VMEM_SHARED((NP, F), jnp.float32),
            [pltpu.SemaphoreType.DMA] * 2,
        ],
        compiler_params=pltpu.CompilerParams(use_tc_tiling_on_sc=False),
    )
    def k(g_hbm, col_hbm, row_hbm, out_hbm, col_v, row_v, buf, table, acc,
          sem_g):
        c = lax.axis_index("c")
        s = lax.axis_index("s")
        wid = s * NC + c

        a0 = s * ROWS_PER_TILE
        pltpu.sync_copy(g_hbm.at[pl.ds(a0, ROWS_PER_TILE)],
                        table.at[pl.ds(a0, ROWS_PER_TILE)])

        z = jnp.zeros((16,), jnp.float32)

        def zero_row(i, _):
            k = i // B
            r = i % B
            buf[k, r, pl.ds(0, 16)] = z
            buf[k, r, pl.ds(16, 16)] = z
            buf[k, r, pl.ds(32, 16)] = z
            buf[k, r, pl.ds(48, 16)] = z
            return _

        lax.fori_loop(0, 2 * B, zero_row, 0)
        for zc in range(5):
            rows = B if zc < 4 else ROWS_PER_TILE - 4 * B
            pltpu.sync_copy(buf.at[zc % 2].at[pl.ds(0, rows)],
                            acc.at[pl.ds(a0 + zc * B, rows)])

        pltpu.sync_copy(col_hbm.at[wid], col_v)
        pltpu.sync_copy(row_hbm.at[wid], row_v)

        plsc.subcore_barrier()

        def fire_gather(j, slot):
            pltpu.async_copy(table.at[col_v.at[j]], buf.at[slot], sem_g[slot])

        def wait_gather(j, slot):
            pltpu.make_async_copy(
                table.at[col_v.at[j]], buf.at[slot], sem_g[slot]).wait()

        fire_gather(0, 0)

        def edge_chunk2(j2, carry):
            j = j2 * 2
            for u in range(2):
                ju = j + u
                jf = ju + 1
                kf = (u + 1) % 2

                @pl.when(jf < CH)
                def _(jf=jf, kf=kf):
                    fire_gather(jf, kf)
                wait_gather(ju, u)
                pltpu.sync_copy(buf.at[u], acc.at[row_v.at[ju]], add=True)
            return carry

        lax.fori_loop(0, CH // 2, edge_chunk2, 0)
        plsc.subcore_barrier()

        r0 = s * ROWS_PER_TILE
        pltpu.sync_copy(acc.at[pl.ds(r0, ROWS_PER_TILE)],
                        out_hbm.at[c].at[pl.ds(r0, ROWS_PER_TILE)])

    return k(g, colp, rowp)


def _mm_body(x_ref, w_ref, o_ref):
    o_ref[:] = jnp.dot(x_ref[:], w_ref[:], preferred_element_type=jnp.float32)


def _combine_body(p_ref, b_ref, w_ref, o_ref):
    h = jnp.maximum(p_ref[0] + p_ref[1] + b_ref[:], 0.0)
    o_ref[:] = jnp.dot(h, w_ref[:], preferred_element_type=jnp.float32)


def _head_body(p_ref, b3_ref, wf1_ref, bf1_ref, wf2_ref, bf2_ref, o_ref):
    h = jnp.maximum(p_ref[0, :N] + p_ref[1, :N] + b3_ref[:], 0.0)
    m = jnp.sum(h, axis=0, keepdims=True) * (1.0 / N)
    a = jnp.dot(m, wf1_ref[:], preferred_element_type=jnp.float32) + bf1_ref[:]
    a = jnp.maximum(a, 0.0)
    z = jnp.sum(a * wf2_ref[:], axis=1, keepdims=True) + bf2_ref[:]
    o_ref[:] = jax.nn.sigmoid(z)


def kernel(x, edge_index, W1, b1, W2, b2, W3, b3, Wf1, bf1, Wf2, bf2):
    f32 = jnp.float32
    col = edge_index[1]
    row = edge_index[0]
    pad_per_w = CH * B - E // NW
    pad_rows = jnp.broadcast_to(
        N + jnp.arange(pad_per_w, dtype=jnp.int32) % (NP - N), (NW, pad_per_w))
    colp = jnp.pad(col.reshape(NW, E // NW),
                   ((0, 0), (0, pad_per_w))).reshape(NW, CH, B)
    rowp = jnp.concatenate(
        [row.reshape(NW, E // NW), pad_rows], axis=1).reshape(NW, CH, B)

    xp = jnp.pad(x, ((0, NP - N), (0, 0)))

    g = pl.pallas_call(
        _mm_body, out_shape=jax.ShapeDtypeStruct((NP, F), f32)
    )(xp, W1.T)

    for W_next, b in ((W2, b1), (W3, b2)):
        p = _segsum_sc(g, colp, rowp)
        g = pl.pallas_call(
            _combine_body, out_shape=jax.ShapeDtypeStruct((NP, F), f32)
        )(p, b.reshape(1, F), W_next.T)

    p = _segsum_sc(g, colp, rowp)
    out = pl.pallas_call(
        _head_body, out_shape=jax.ShapeDtypeStruct((1, 1), f32)
    )(p, b3.reshape(1, F), Wf1.T, bf1.reshape(1, 32), Wf2, bf2.reshape(1, 1))
    return out.reshape(1)
